# baseline (speedup 1.0000x reference)
import numpy as np
import jax
import jax.numpy as jnp
from jax.experimental import pallas as pl

_ROW = 6
_SQRT2 = float(np.sqrt(2.0))

_RUNS = [(i * _ROW + i, _ROW - i) for i in range(_ROW)]


def _tc_body(x_ref, o_ref):
    lane = jax.lax.broadcasted_iota(jnp.int32, (1, 36), 1)
    scale = jnp.where(lane % _ROW > lane // _ROW, _SQRT2, 1.0).astype(jnp.float32)
    x = jnp.abs(x_ref[...]) * scale
    o_ref[...] = jnp.concatenate(
        [x[:, s : s + l] for (s, l) in _RUNS], axis=1
    )


def kernel(input_ts):
    n = int(np.prod(input_ts.shape[:-2]))
    x2d = input_ts.reshape(n, 36)
    blk = 2048
    out = pl.pallas_call(
        _tc_body,
        grid=(n // blk,),
        in_specs=[pl.BlockSpec((blk, 36), lambda i: (i, 0))],
        out_specs=pl.BlockSpec((blk, 21), lambda i: (i, 0)),
        out_shape=jax.ShapeDtypeStruct((n, 21), jnp.float32),
    )(x2d)
    return out.reshape(*input_ts.shape[:-2], 21, 1)


# speedup vs baseline: 14.4558x; 14.4558x over previous
import numpy as np
import jax
import jax.numpy as jnp
from jax import lax
from jax.experimental import pallas as pl
from jax.experimental.pallas import tpu as pltpu
from jax.experimental.pallas import tpu_sc as plsc

_SQRT2 = float(np.sqrt(2.0))

_N_IN_ROWS = 115200
_N_OUT_ROWS = 67200
_N_TILES = 32
_CHUNK = 168
_CHUNK_PAD = 176
_N_CHUNKS = _N_OUT_ROWS // _CHUNK
_MAX_PER_TILE = -(-_N_CHUNKS // _N_TILES)

_CUM = [0, 6, 11, 15, 18, 20]
_DIAG_T = set(_CUM)


def _sc_body(x_hbm, o_hbm, idx_buf, row_buf, sem):
    cid = lax.axis_index("c")
    sid = lax.axis_index("s")
    wid = sid * 2 + cid

    def process(c):
        one = jnp.full((16,), 1, jnp.int32)
        cv = one * c
        qv = (cv * 1639) >> 13
        rphase = cv - qv * 5
        for v in range(_CHUNK_PAD // 16):
            p = v * 16 + lax.iota(jnp.int32, 16)
            u = rphase * _CHUNK + p
            blkoff = (u * 1249) >> 18
            w = u - blkoff * 210
            d = jnp.where(w >= 105, 1, 0)
            r2 = w - d * 105
            e = (jnp.where(r2 >= 21, 1, 0) + jnp.where(r2 >= 42, 1, 0)
                 + jnp.where(r2 >= 63, 1, 0) + jnp.where(r2 >= 84, 1, 0))
            t = r2 - e * 21
            i = (jnp.where(t >= 6, 1, 0) + jnp.where(t >= 11, 1, 0)
                 + jnp.where(t >= 15, 1, 0) + jnp.where(t >= 18, 1, 0)
                 + jnp.where(t >= 20, 1, 0))
            j = t - ((i * (13 - i)) >> 1) + i
            m = i * 6 + j
            src = qv * 1440 + blkoff * 360 + e * 72 + m * 2 + d
            idx_buf[pl.ds(v * 16, 16)] = jnp.minimum(src, _N_IN_ROWS - 1)

        pltpu.async_copy(x_hbm.at[idx_buf], row_buf, sem).wait()

        def group(g, _):
            for t in range(21):
                s = 1.0 if t in _DIAG_T else _SQRT2
                for v in range(8):
                    sl = pl.ds(v * 16, 16)
                    row_buf[g * 21 + t, sl] = (
                        jnp.abs(row_buf[g * 21 + t, sl]) * s
                    )
            return 0

        lax.fori_loop(0, _CHUNK // 21, group, 0)

        pltpu.async_copy(
            row_buf.at[pl.ds(0, _CHUNK)],
            o_hbm.at[pl.ds(c * _CHUNK, _CHUNK)],
            sem,
        ).wait()

    def tile_loop(jj, _):
        c = wid + _N_TILES * jj

        @pl.when(c < _N_CHUNKS)
        def _():
            process(c)

        return 0

    lax.fori_loop(0, _MAX_PER_TILE, tile_loop, 0)


def kernel(input_ts):
    x2d = input_ts.transpose(0, 2, 4, 5, 6, 3, 1).reshape(_N_IN_ROWS, 128)

    mesh = plsc.VectorSubcoreMesh(core_axis_name="c", subcore_axis_name="s")
    out2d = pl.kernel(
        _sc_body,
        out_type=jax.ShapeDtypeStruct((_N_OUT_ROWS, 128), jnp.float32),
        mesh=mesh,
        scratch_types=[
            pltpu.VMEM((_CHUNK_PAD,), jnp.int32),
            pltpu.VMEM((_CHUNK_PAD, 128), jnp.float32),
            pltpu.SemaphoreType.DMA,
        ],
    )(x2d)

    out = out2d.reshape(64, 5, 2, 5, 21, 128)
    return out.transpose(0, 5, 1, 2, 3, 4)[..., None]


# speedup vs baseline: 16.9322x; 1.1713x over previous
import numpy as np
import jax
import jax.numpy as jnp
from jax import lax
from jax.experimental import pallas as pl
from jax.experimental.pallas import tpu as pltpu
from jax.experimental.pallas import tpu_sc as plsc

_SQRT2 = float(np.sqrt(2.0))

_N_TBL_ROWS = 57600
_N_OUT_ROWS = 67200
_N_TILES = 32
_BLK_OUT = 210
_BLK_UNITS = 105
_UNITS_PAD = 112
_BLKS_PER_TILE = 10

_CUM = [0, 6, 11, 15, 18, 20]
_DIAG_T = set(_CUM)


def _sc_body(x_hbm, o_hbm, idx_buf, gbuf, sbuf, gs0, gs1, ss0, ss1):
    gsem = [gs0, gs1]
    ssem = [ss0, ss1]
    cid = lax.axis_index("c")
    sid = lax.axis_index("s")
    wid = sid * 2 + cid
    blk0 = wid * _BLKS_PER_TILE

    def fill_idx(blk, b):
        for v in range(_UNITS_PAD // 16):
            p = v * 16 + lax.iota(jnp.int32, 16)
            p = jnp.minimum(p, _BLK_UNITS - 1)
            e = (jnp.where(p >= 21, 1, 0) + jnp.where(p >= 42, 1, 0)
                 + jnp.where(p >= 63, 1, 0) + jnp.where(p >= 84, 1, 0))
            t = p - e * 21
            i = (jnp.where(t >= 6, 1, 0) + jnp.where(t >= 11, 1, 0)
                 + jnp.where(t >= 15, 1, 0) + jnp.where(t >= 18, 1, 0)
                 + jnp.where(t >= 20, 1, 0))
            j = t - ((i * (13 - i)) >> 1) + i
            m = i * 6 + j
            idx_buf[b, 0, pl.ds(v * 16, 16)] = blk * 180 + e * 36 + m

    def gather(b):
        return pltpu.make_async_copy(
            x_hbm.at[idx_buf.at[b, 0]], gbuf.at[b], gsem[b]
        )

    def scatter(blk, b):
        return pltpu.make_async_copy(
            sbuf.at[b],
            o_hbm.at[pl.ds(blk * _BLK_OUT, _BLK_OUT)],
            ssem[b],
        )

    fill_idx(blk0, 0)
    gather(0).start()

    def step(j, b):
        blk = blk0 + j

        @pl.when(j < _BLKS_PER_TILE - 1)
        def _():
            @pl.when(j >= 1)
            def _():
                scatter(blk - 1, 1 - b).wait()

            fill_idx(blk + 1, 1 - b)
            gather(1 - b).start()

        gather(b).wait()

        def e_loop(e, _):
            for t in range(21):
                s = 1.0 if t in _DIAG_T else _SQRT2
                for d in range(2):
                    for v in range(8):
                        src = pl.ds(d * 128 + v * 16, 16)
                        dst = pl.ds(v * 16, 16)
                        sbuf[b, d * 105 + e * 21 + t, 0, dst] = (
                            jnp.abs(gbuf[b, e * 21 + t, 0, src]) * s
                        )
            return 0

        lax.fori_loop(0, 5, e_loop, 0)
        scatter(blk, b).start()

    def outer(jo, _):
        for bb in range(2):
            step(jo * 2 + bb, bb)
        return 0

    lax.fori_loop(0, _BLKS_PER_TILE // 2, outer, 0)

    scatter(blk0 + _BLKS_PER_TILE - 2, 0).wait()
    scatter(blk0 + _BLKS_PER_TILE - 1, 1).wait()


def kernel(input_ts):
    x_tbl = input_ts.transpose(0, 2, 4, 5, 6, 3, 1).reshape(
        _N_TBL_ROWS, 1, 256
    )

    mesh = plsc.VectorSubcoreMesh(core_axis_name="c", subcore_axis_name="s")
    out3d = pl.kernel(
        _sc_body,
        out_type=jax.ShapeDtypeStruct((_N_OUT_ROWS, 1, 128), jnp.float32),
        mesh=mesh,
        scratch_types=[
            pltpu.VMEM((2, 1, _UNITS_PAD), jnp.int32),
            pltpu.VMEM((2, _UNITS_PAD, 1, 256), jnp.float32),
            pltpu.VMEM((2, _BLK_OUT, 1, 128), jnp.float32),
            pltpu.SemaphoreType.DMA,
            pltpu.SemaphoreType.DMA,
            pltpu.SemaphoreType.DMA,
            pltpu.SemaphoreType.DMA,
        ],
    )(x_tbl)

    out = out3d.reshape(64, 5, 2, 5, 21, 1, 128)
    return out.transpose(0, 6, 1, 2, 3, 4, 5)


# speedup vs baseline: 39.5234x; 2.3342x over previous
import numpy as np
import jax
import jax.numpy as jnp
from jax import lax
from jax.experimental import pallas as pl
from jax.experimental.pallas import tpu as pltpu
from jax.experimental.pallas import tpu_sc as plsc

_SQRT2 = float(np.sqrt(2.0))

_N_IN_ROWS = 115200
_N_OUT_ROWS = 67200
_N_TILES = 32
_CHUNK = 168
_CHUNK_PAD = 176
_N_CHUNKS = _N_OUT_ROWS // _CHUNK
_MAX_PER_TILE = -(-_N_CHUNKS // _N_TILES)

_CUM = [0, 6, 11, 15, 18, 20]
_DIAG_T = set(_CUM)


_NBUF = 4


def _sc_body(x_hbm, o_hbm, idx_buf, row_buf, gs0, gs1, gs2, gs3, ss0, ss1, ss2, ss3):
    gsem = [gs0, gs1, gs2, gs3]
    ssem = [ss0, ss1, ss2, ss3]
    cid = lax.axis_index("c")
    sid = lax.axis_index("s")
    wid = sid * 2 + cid

    def fill_idx(c, b):
        one = jnp.full((16,), 1, jnp.int32)
        cv = one * c
        qv = (cv * 1639) >> 13
        rphase = cv - qv * 5
        for v in range(_CHUNK_PAD // 16):
            p = v * 16 + lax.iota(jnp.int32, 16)
            u = rphase * _CHUNK + p
            blkoff = (u * 1249) >> 18
            w = u - blkoff * 210
            d = jnp.where(w >= 105, 1, 0)
            r2 = w - d * 105
            e = (jnp.where(r2 >= 21, 1, 0) + jnp.where(r2 >= 42, 1, 0)
                 + jnp.where(r2 >= 63, 1, 0) + jnp.where(r2 >= 84, 1, 0))
            t = r2 - e * 21
            i = (jnp.where(t >= 6, 1, 0) + jnp.where(t >= 11, 1, 0)
                 + jnp.where(t >= 15, 1, 0) + jnp.where(t >= 18, 1, 0)
                 + jnp.where(t >= 20, 1, 0))
            j = t - ((i * (13 - i)) >> 1) + i
            m = i * 6 + j
            src = qv * 1440 + blkoff * 360 + e * 72 + m * 2 + d
            idx_buf[b, 0, pl.ds(v * 16, 16)] = jnp.minimum(src, _N_IN_ROWS - 1)

    def gather(b):
        return pltpu.make_async_copy(
            x_hbm.at[idx_buf.at[b, 0]], row_buf.at[b], gsem[b]
        )

    def scatter(c, b):
        return pltpu.make_async_copy(
            row_buf.at[b, pl.ds(0, _CHUNK)],
            o_hbm.at[pl.ds(c * _CHUNK, _CHUNK)],
            ssem[b],
        )

    fill_idx(wid, 0)
    gather(0).start()
    fill_idx(wid + _N_TILES, 1)
    gather(1).start()

    def step(j, b):
        c = wid + _N_TILES * j

        bp = (b + 2) % _NBUF
        cp = wid + _N_TILES * (j + 2)

        @pl.when(cp < _N_CHUNKS)
        def _():
            @pl.when(j >= 2)
            def _():
                scatter(wid + _N_TILES * (j - 2), bp).wait()

            fill_idx(cp, bp)
            gather(bp).start()

        @pl.when(c < _N_CHUNKS)
        def _():
            gather(b).wait()

            def group(g, _):
                for t in range(21):
                    s = 1.0 if t in _DIAG_T else _SQRT2
                    for v in range(8):
                        sl = pl.ds(v * 16, 16)
                        row_buf[b, g * 21 + t, 0, sl] = (
                            jnp.abs(row_buf[b, g * 21 + t, 0, sl]) * s
                        )
                return 0

            lax.fori_loop(0, _CHUNK // 21, group, 0)
            scatter(c, b).start()

    def outer(jo, _):
        for bb in range(_NBUF):
            step(jo * _NBUF + bb, bb)
        return 0

    n_outer = -(-(_MAX_PER_TILE + 1) // _NBUF)
    lax.fori_loop(0, n_outer, outer, 0)

    for j_last in range(_MAX_PER_TILE):
        c_last = wid + _N_TILES * j_last
        c_nxt = wid + _N_TILES * (j_last + _NBUF)

        @pl.when((c_last < _N_CHUNKS) & (c_nxt >= _N_CHUNKS))
        def _():
            scatter(c_last, j_last % _NBUF).wait()


def kernel(input_ts):
    x3d = input_ts.transpose(0, 2, 4, 5, 6, 3, 1).reshape(_N_IN_ROWS, 1, 128)

    mesh = plsc.VectorSubcoreMesh(core_axis_name="c", subcore_axis_name="s")
    out3d = pl.kernel(
        _sc_body,
        out_type=jax.ShapeDtypeStruct((_N_OUT_ROWS, 1, 128), jnp.float32),
        mesh=mesh,
        scratch_types=[
            pltpu.VMEM((_NBUF, 1, _CHUNK_PAD), jnp.int32),
            pltpu.VMEM((_NBUF, _CHUNK_PAD, 1, 128), jnp.float32),
            pltpu.SemaphoreType.DMA,
            pltpu.SemaphoreType.DMA,
            pltpu.SemaphoreType.DMA,
            pltpu.SemaphoreType.DMA,
            pltpu.SemaphoreType.DMA,
            pltpu.SemaphoreType.DMA,
            pltpu.SemaphoreType.DMA,
            pltpu.SemaphoreType.DMA,
        ],
    )(x3d)

    out = out3d.reshape(64, 5, 2, 5, 21, 1, 128)
    return out.transpose(0, 6, 1, 2, 3, 4, 5)


# speedup vs baseline: 40.4494x; 1.0234x over previous
import numpy as np
import jax
import jax.numpy as jnp
from jax import lax
from jax.experimental import pallas as pl
from jax.experimental.pallas import tpu as pltpu
from jax.experimental.pallas import tpu_sc as plsc

_SQRT2 = float(np.sqrt(2.0))

_N_TBL_ROWS = 57600
_N_OUT_ROWS = 67200
_N_TILES = 32
_BLK_OUT = 210
_BLK_UNITS = 105
_UNITS_PAD = 112
_BLKS_PER_TILE = 10
_NBUF = 4

_CUM = [0, 6, 11, 15, 18, 20]
_DIAG_T = set(_CUM)


def _sc_body(x_hbm, o_hbm, idx_buf, gbuf, gs0, gs1, gs2, gs3,
             ss0, ss1, ss2, ss3):
    gsem = [gs0, gs1, gs2, gs3]
    ssem = [ss0, ss1, ss2, ss3]
    cid = lax.axis_index("c")
    sid = lax.axis_index("s")
    wid = sid * 2 + cid
    blk0 = wid * _BLKS_PER_TILE

    def fill_idx(blk, b):
        for v in range(_UNITS_PAD // 16):
            p = v * 16 + lax.iota(jnp.int32, 16)
            p = jnp.minimum(p, _BLK_UNITS - 1)
            e = (jnp.where(p >= 21, 1, 0) + jnp.where(p >= 42, 1, 0)
                 + jnp.where(p >= 63, 1, 0) + jnp.where(p >= 84, 1, 0))
            t = p - e * 21
            i = (jnp.where(t >= 6, 1, 0) + jnp.where(t >= 11, 1, 0)
                 + jnp.where(t >= 15, 1, 0) + jnp.where(t >= 18, 1, 0)
                 + jnp.where(t >= 20, 1, 0))
            j = t - ((i * (13 - i)) >> 1) + i
            m = i * 6 + j
            idx_buf[b, 0, pl.ds(v * 16, 16)] = blk * 180 + e * 36 + m

    def gather(b):
        return pltpu.make_async_copy(
            x_hbm.at[idx_buf.at[b, 0]], gbuf.at[b], gsem[b]
        )

    def scatter(blk, b, d):
        return pltpu.make_async_copy(
            gbuf.at[b, pl.ds(0, _BLK_UNITS), pl.ds(d, 1)],
            o_hbm.at[pl.ds(blk * _BLK_OUT + d * _BLK_UNITS, _BLK_UNITS)],
            ssem[b],
        )

    fill_idx(blk0, 0)
    gather(0).start()
    fill_idx(blk0 + 1, 1)
    gather(1).start()

    def step(j, b):
        blk = blk0 + j

        bp = (b + 2) % _NBUF

        @pl.when(j + 2 < _BLKS_PER_TILE)
        def _():
            @pl.when(j >= 2)
            def _():
                scatter(blk - 2, bp, 0).wait()
                scatter(blk - 2, bp, 1).wait()

            fill_idx(blk + 2, bp)
            gather(bp).start()

        @pl.when(j < _BLKS_PER_TILE)
        def _():
            gather(b).wait()

            def e_loop(e, _):
                for t in range(21):
                    s = 1.0 if t in _DIAG_T else _SQRT2
                    for d in range(2):
                        for v in range(8):
                            sl = pl.ds(v * 16, 16)
                            gbuf[b, e * 21 + t, d, sl] = (
                                jnp.abs(gbuf[b, e * 21 + t, d, sl]) * s
                            )
                return 0

            lax.fori_loop(0, 5, e_loop, 0)
            scatter(blk, b, 0).start()
            scatter(blk, b, 1).start()

    def outer(jo, _):
        for bb in range(_NBUF):
            step(jo * _NBUF + bb, bb)
        return 0

    lax.fori_loop(0, -(-(_BLKS_PER_TILE + 2) // _NBUF), outer, 0)

    for j_last in range(max(0, _BLKS_PER_TILE - _NBUF), _BLKS_PER_TILE):
        scatter(blk0 + j_last, j_last % _NBUF, 0).wait()
        scatter(blk0 + j_last, j_last % _NBUF, 1).wait()


def kernel(input_ts):
    x_tbl = input_ts.transpose(0, 2, 4, 5, 6, 3, 1).reshape(
        _N_TBL_ROWS, 2, 128
    )

    mesh = plsc.VectorSubcoreMesh(core_axis_name="c", subcore_axis_name="s")
    out3d = pl.kernel(
        _sc_body,
        out_type=jax.ShapeDtypeStruct((_N_OUT_ROWS, 1, 128), jnp.float32),
        mesh=mesh,
        scratch_types=[
            pltpu.VMEM((_NBUF, 1, _UNITS_PAD), jnp.int32),
            pltpu.VMEM((_NBUF, _UNITS_PAD, 2, 128), jnp.float32),
            pltpu.SemaphoreType.DMA,
            pltpu.SemaphoreType.DMA,
            pltpu.SemaphoreType.DMA,
            pltpu.SemaphoreType.DMA,
            pltpu.SemaphoreType.DMA,
            pltpu.SemaphoreType.DMA,
            pltpu.SemaphoreType.DMA,
            pltpu.SemaphoreType.DMA,
        ],
    )(x_tbl)

    out = out3d.reshape(64, 5, 2, 5, 21, 1, 128)
    return out.transpose(0, 6, 1, 2, 3, 4, 5)


# speedup vs baseline: 40.8353x; 1.0095x over previous
import numpy as np
import jax
import jax.numpy as jnp
from jax import lax
from jax.experimental import pallas as pl
from jax.experimental.pallas import tpu as pltpu
from jax.experimental.pallas import tpu_sc as plsc

_SQRT2 = float(np.sqrt(2.0))

_N_IN_ROWS = 115200
_N_OUT_ROWS = 67200
_N_TILES = 32
_CHUNK = 168
_CHUNK_PAD = 176
_N_CHUNKS = _N_OUT_ROWS // _CHUNK
_MAX_PER_TILE = -(-_N_CHUNKS // _N_TILES)

_CUM = [0, 6, 11, 15, 18, 20]
_DIAG_T = set(_CUM)


_NBUF = 3


def _sc_body(x_hbm, o_hbm, idx_buf, row_buf, gs0, gs1, gs2, ss0, ss1, ss2):
    gsem = [gs0, gs1, gs2]
    ssem = [ss0, ss1, ss2]
    cid = lax.axis_index("c")
    sid = lax.axis_index("s")
    wid = sid * 2 + cid

    def fill_idx(c, b):
        one = jnp.full((16,), 1, jnp.int32)
        cv = one * c
        qv = (cv * 1639) >> 13
        rphase = cv - qv * 5
        for v in range(_CHUNK_PAD // 16):
            p = v * 16 + lax.iota(jnp.int32, 16)
            u = rphase * _CHUNK + p
            blkoff = (u * 1249) >> 18
            w = u - blkoff * 210
            d = jnp.where(w >= 105, 1, 0)
            r2 = w - d * 105
            e = (jnp.where(r2 >= 21, 1, 0) + jnp.where(r2 >= 42, 1, 0)
                 + jnp.where(r2 >= 63, 1, 0) + jnp.where(r2 >= 84, 1, 0))
            t = r2 - e * 21
            i = (jnp.where(t >= 6, 1, 0) + jnp.where(t >= 11, 1, 0)
                 + jnp.where(t >= 15, 1, 0) + jnp.where(t >= 18, 1, 0)
                 + jnp.where(t >= 20, 1, 0))
            j = t - ((i * (13 - i)) >> 1) + i
            m = i * 6 + j
            src = qv * 1440 + blkoff * 360 + e * 72 + m * 2 + d
            idx_buf[b, 0, pl.ds(v * 16, 16)] = jnp.minimum(src, _N_IN_ROWS - 1)

    def gather(b):
        return pltpu.make_async_copy(
            x_hbm.at[idx_buf.at[b, 0]], row_buf.at[b], gsem[b]
        )

    def scatter(c, b):
        return pltpu.make_async_copy(
            row_buf.at[b, pl.ds(0, _CHUNK)],
            o_hbm.at[pl.ds(c * _CHUNK, _CHUNK)],
            ssem[b],
        )

    fill_idx(wid, 0)
    gather(0).start()
    fill_idx(wid + _N_TILES, 1)
    gather(1).start()

    def step(j, b):
        c = wid + _N_TILES * j

        bp = (b + 2) % _NBUF
        cp = wid + _N_TILES * (j + 2)

        @pl.when(cp < _N_CHUNKS)
        def _():
            @pl.when(j >= 1)
            def _():
                scatter(wid + _N_TILES * (j - 1), bp).wait()

            fill_idx(cp, bp)
            gather(bp).start()

        @pl.when(c < _N_CHUNKS)
        def _():
            gather(b).wait()

            def group(g, _):
                for t in range(21):
                    s = 1.0 if t in _DIAG_T else _SQRT2
                    for v in range(8):
                        sl = pl.ds(v * 16, 16)
                        row_buf[b, g * 21 + t, 0, sl] = (
                            jnp.abs(row_buf[b, g * 21 + t, 0, sl]) * s
                        )
                return 0

            lax.fori_loop(0, _CHUNK // 21, group, 0)
            scatter(c, b).start()

    def outer(jo, _):
        for bb in range(_NBUF):
            step(jo * _NBUF + bb, bb)
        return 0

    n_outer = -(-(_MAX_PER_TILE + 1) // _NBUF)
    lax.fori_loop(0, n_outer, outer, 0)

    for j_last in range(_MAX_PER_TILE):
        c_last = wid + _N_TILES * j_last
        c_nxt = wid + _N_TILES * (j_last + _NBUF)

        @pl.when((c_last < _N_CHUNKS) & (c_nxt >= _N_CHUNKS))
        def _():
            scatter(c_last, j_last % _NBUF).wait()


def kernel(input_ts):
    x3d = input_ts.transpose(0, 2, 4, 5, 6, 3, 1).reshape(_N_IN_ROWS, 1, 128)

    mesh = plsc.VectorSubcoreMesh(core_axis_name="c", subcore_axis_name="s")
    out3d = pl.kernel(
        _sc_body,
        out_type=jax.ShapeDtypeStruct((_N_OUT_ROWS, 1, 128), jnp.float32),
        mesh=mesh,
        scratch_types=[
            pltpu.VMEM((_NBUF, 1, _CHUNK_PAD), jnp.int32),
            pltpu.VMEM((_NBUF, _CHUNK_PAD, 1, 128), jnp.float32),
            pltpu.SemaphoreType.DMA,
            pltpu.SemaphoreType.DMA,
            pltpu.SemaphoreType.DMA,
            pltpu.SemaphoreType.DMA,
            pltpu.SemaphoreType.DMA,
            pltpu.SemaphoreType.DMA,
        ],
    )(x3d)

    out = out3d.reshape(64, 5, 2, 5, 21, 1, 128)
    return out.transpose(0, 6, 1, 2, 3, 4, 5)
